# baseline (speedup 1.0000x reference)
import jax
import jax.numpy as jnp
from jax.experimental import pallas as pl
from jax.experimental.pallas import tpu as pltpu

_BM = 1024


def _encoder_body(x_ref, w1_ref, b1_ref, w2_ref, b2_ref, o_ref):
    h = jax.lax.dot_general(
        x_ref[...].astype(jnp.bfloat16), w1_ref[...].astype(jnp.bfloat16),
        dimension_numbers=(((1,), (0,)), ((), ())),
        preferred_element_type=jnp.float32,
    )
    h = jnp.maximum(h + b1_ref[...], 0.0)
    z = jax.lax.dot_general(
        h.astype(jnp.bfloat16), w2_ref[...].astype(jnp.bfloat16),
        dimension_numbers=(((1,), (0,)), ((), ())),
        preferred_element_type=jnp.float32,
    )
    o_ref[...] = z + b2_ref[...]


def kernel(inputs, enc_w1, enc_b1, enc_w2, enc_b2,
           dec_w1, dec_b1, dec_w2, dec_b2, prior):
    del dec_w1, dec_b1, dec_w2, dec_b2, prior
    b, feat = inputs.shape
    hid = enc_w1.shape[1]
    code = enc_w2.shape[1]
    grid = (b // _BM,)
    out = pl.pallas_call(
        _encoder_body,
        grid=grid,
        in_specs=[
            pl.BlockSpec((_BM, feat), lambda i: (i, 0)),
            pl.BlockSpec((feat, hid), lambda i: (0, 0)),
            pl.BlockSpec((1, hid), lambda i: (0, 0)),
            pl.BlockSpec((hid, code), lambda i: (0, 0)),
            pl.BlockSpec((1, code), lambda i: (0, 0)),
        ],
        out_specs=pl.BlockSpec((_BM, code), lambda i: (i, 0)),
        out_shape=jax.ShapeDtypeStruct((b, code), jnp.float32),
        compiler_params=pltpu.CompilerParams(
            dimension_semantics=("arbitrary",),
        ),
    )(inputs, enc_w1, enc_b1.reshape(1, hid), enc_w2, enc_b2.reshape(1, code))
    return out


# speedup vs baseline: 1.0062x; 1.0062x over previous
import jax
import jax.numpy as jnp
from jax.experimental import pallas as pl
from jax.experimental.pallas import tpu as pltpu

_BM = 1024


def _encoder_body(x_ref, w1_ref, b1_ref, w2_ref, b2_ref, o_ref):
    h = jax.lax.dot_general(
        x_ref[...].astype(jnp.bfloat16), w1_ref[...].astype(jnp.bfloat16),
        dimension_numbers=(((1,), (0,)), ((), ())),
        preferred_element_type=jnp.float32,
    )
    h = jnp.maximum(h + b1_ref[...], 0.0)
    z = jax.lax.dot_general(
        h.astype(jnp.bfloat16), w2_ref[...].astype(jnp.bfloat16),
        dimension_numbers=(((1,), (0,)), ((), ())),
        preferred_element_type=jnp.float32,
    )
    o_ref[...] = z + b2_ref[...]


def kernel(inputs, enc_w1, enc_b1, enc_w2, enc_b2,
           dec_w1, dec_b1, dec_w2, dec_b2, prior):
    del dec_w1, dec_b1, dec_w2, dec_b2, prior
    b, feat = inputs.shape
    hid = enc_w1.shape[1]
    code = enc_w2.shape[1]
    grid = (b // _BM,)
    out = pl.pallas_call(
        _encoder_body,
        grid=grid,
        in_specs=[
            pl.BlockSpec((_BM, feat), lambda i: (i, 0)),
            pl.BlockSpec((feat, hid), lambda i: (0, 0)),
            pl.BlockSpec((1, hid), lambda i: (0, 0)),
            pl.BlockSpec((hid, code), lambda i: (0, 0)),
            pl.BlockSpec((1, code), lambda i: (0, 0)),
        ],
        out_specs=pl.BlockSpec((_BM, code), lambda i: (i, 0)),
        out_shape=jax.ShapeDtypeStruct((b, code), jnp.float32),
        compiler_params=pltpu.CompilerParams(
            dimension_semantics=("parallel",),
        ),
    )(inputs, enc_w1, enc_b1.reshape(1, hid), enc_w2, enc_b2.reshape(1, code))
    return out


# speedup vs baseline: 1.0996x; 1.0928x over previous
import jax
import jax.numpy as jnp
from jax.experimental import pallas as pl
from jax.experimental.pallas import tpu as pltpu

_BM = 2048


def _encoder_body(x_ref, w1_ref, b1_ref, w2_ref, b2_ref, o_ref):
    h = jax.lax.dot_general(
        x_ref[...].astype(jnp.bfloat16), w1_ref[...].astype(jnp.bfloat16),
        dimension_numbers=(((1,), (0,)), ((), ())),
        preferred_element_type=jnp.float32,
    )
    h = jnp.maximum(h + b1_ref[...], 0.0)
    z = jax.lax.dot_general(
        h.astype(jnp.bfloat16), w2_ref[...].astype(jnp.bfloat16),
        dimension_numbers=(((1,), (0,)), ((), ())),
        preferred_element_type=jnp.float32,
    )
    o_ref[...] = z + b2_ref[...]


def kernel(inputs, enc_w1, enc_b1, enc_w2, enc_b2,
           dec_w1, dec_b1, dec_w2, dec_b2, prior):
    del dec_w1, dec_b1, dec_w2, dec_b2, prior
    b, feat = inputs.shape
    hid = enc_w1.shape[1]
    code = enc_w2.shape[1]
    grid = (b // _BM,)
    out = pl.pallas_call(
        _encoder_body,
        grid=grid,
        in_specs=[
            pl.BlockSpec((_BM, feat), lambda i: (i, 0)),
            pl.BlockSpec((feat, hid), lambda i: (0, 0)),
            pl.BlockSpec((1, hid), lambda i: (0, 0)),
            pl.BlockSpec((hid, code), lambda i: (0, 0)),
            pl.BlockSpec((1, code), lambda i: (0, 0)),
        ],
        out_specs=pl.BlockSpec((_BM, code), lambda i: (i, 0)),
        out_shape=jax.ShapeDtypeStruct((b, code), jnp.float32),
        compiler_params=pltpu.CompilerParams(
            dimension_semantics=("parallel",),
        ),
    )(inputs, enc_w1, enc_b1.reshape(1, hid), enc_w2, enc_b2.reshape(1, code))
    return out
